# baseline (speedup 1.0000x reference)
import functools

import jax
import jax.numpy as jnp
from jax import lax
from jax.experimental import pallas as pl
from jax.experimental.pallas import tpu as pltpu
from jax.experimental.pallas import tpu_sc as plsc

D_MODEL = 1024
NUM_BANDS = 5
BATCH = 16384

NC, NS = 2, 16
NW = NC * NS
B_TOTAL = BATCH * NUM_BANDS
BPW = B_TOTAL // NW
CHUNK = 40
NCHUNK = BPW // CHUNK
NBUF = 2

_LO = (0.5, 4.0, 8.0, 13.0, 30.0)
_HI = (4.0, 8.0, 13.0, 30.0, 100.0)


def _combine_body(emb_ref, wt_ref, b_ref, lo_ref, hi_ref, out_ref):
    w0 = wt_ref[0:1, :]
    w1 = wt_ref[1:2, :]
    out_ref[:, :] = (
        emb_ref[:, :] + lo_ref[:, :] * w0 + hi_ref[:, :] * w1
        + b_ref[:].reshape(1, D_MODEL)
    )


def _combine(emb_table, freq_wt, freq_b):
    lo = jnp.array(_LO, dtype=jnp.float32).reshape(NUM_BANDS, 1)
    hi = jnp.array(_HI, dtype=jnp.float32).reshape(NUM_BANDS, 1)
    return pl.pallas_call(
        _combine_body,
        out_shape=jax.ShapeDtypeStruct((NUM_BANDS, D_MODEL), jnp.float32),
    )(emb_table, freq_wt, freq_b, lo, hi)


_MESH = plsc.VectorSubcoreMesh(core_axis_name="c", subcore_axis_name="s")


@functools.partial(
    pl.kernel,
    out_type=jax.ShapeDtypeStruct((B_TOTAL, D_MODEL), jnp.float32),
    mesh=_MESH,
    scratch_types=[
        pltpu.VMEM((NCHUNK, CHUNK), jnp.int32),
        pltpu.VMEM((NBUF, CHUNK, D_MODEL), jnp.float32),
        pltpu.SemaphoreType.DMA,
        pltpu.SemaphoreType.DMA,
    ],
)
def _gather(table_hbm, idx_hbm, out_hbm, idx_v, rows_v, gsem, ssem):
    wid = lax.axis_index("s") * NC + lax.axis_index("c")
    base = wid * BPW

    pltpu.sync_copy(idx_hbm.at[wid], idx_v)

    def start_gather(g, slot):
        pltpu.async_copy(table_hbm.at[idx_v.at[g]], rows_v.at[slot], gsem)

    def wait_gather(slot):
        pltpu.make_async_copy(
            out_hbm.at[pl.ds(0, CHUNK)], rows_v.at[slot], gsem
        ).wait()

    def start_scatter(g, slot):
        pltpu.async_copy(
            rows_v.at[slot], out_hbm.at[pl.ds(base + g * CHUNK, CHUNK)], ssem
        )

    def wait_scatter(slot):
        pltpu.make_async_copy(
            rows_v.at[slot], out_hbm.at[pl.ds(0, CHUNK)], ssem
        ).wait()

    start_gather(0, 0)

    def body(g, _):
        slot = lax.rem(g, NBUF)
        wait_gather(slot)
        start_scatter(g, slot)

        @pl.when(g >= 1)
        def _():
            wait_scatter(1 - slot)

        @pl.when(g + 1 < NCHUNK)
        def _():
            start_gather(g + 1, 1 - slot)

        return 0

    lax.fori_loop(0, NCHUNK, body, 0)
    wait_scatter((NCHUNK - 1) % NBUF)


def kernel(band_indices, emb_table, freq_w, freq_b):
    table = _combine(emb_table, freq_w.T, freq_b)
    idx = band_indices.reshape(NW, NCHUNK, CHUNK)
    out = _gather(table, idx)
    return out.reshape(BATCH, NUM_BANDS, D_MODEL)


# speedup vs baseline: 1.7052x; 1.7052x over previous
import functools

import jax
import jax.numpy as jnp
from jax import lax
from jax.experimental import pallas as pl
from jax.experimental.pallas import tpu as pltpu
from jax.experimental.pallas import tpu_sc as plsc

D_MODEL = 1024
NUM_BANDS = 5
BATCH = 16384

NC, NS = 2, 16
NW = NC * NS
BPW = BATCH // NW
BCH = 4
NGRP = BPW // (2 * BCH)
NBUF = 3
LANES = 16
NVEC = D_MODEL // LANES

_LO = (0.5, 4.0, 8.0, 13.0, 30.0)
_HI = (4.0, 8.0, 13.0, 30.0, 100.0)


def _combine_body(emb_ref, wt_ref, b_ref, lo_ref, hi_ref, out_ref):
    w0 = wt_ref[0:1, :]
    w1 = wt_ref[1:2, :]
    out_ref[:, :] = (
        emb_ref[:, :] + lo_ref[:, :] * w0 + hi_ref[:, :] * w1
        + b_ref[:].reshape(1, D_MODEL)
    )


def _combine(emb_table, freq_wt, freq_b):
    lo = jnp.array(_LO, dtype=jnp.float32).reshape(NUM_BANDS, 1)
    hi = jnp.array(_HI, dtype=jnp.float32).reshape(NUM_BANDS, 1)
    return pl.pallas_call(
        _combine_body,
        out_shape=jax.ShapeDtypeStruct((NUM_BANDS, D_MODEL), jnp.float32),
    )(emb_table, freq_wt, freq_b, lo, hi)


_MESH = plsc.VectorSubcoreMesh(core_axis_name="c", subcore_axis_name="s")


@functools.partial(
    pl.kernel,
    out_type=jax.ShapeDtypeStruct((BATCH, NUM_BANDS, D_MODEL), jnp.float32),
    mesh=_MESH,
    scratch_types=[
        pltpu.VMEM((NUM_BANDS * D_MODEL,), jnp.float32),
        pltpu.VMEM((BPW * NUM_BANDS,), jnp.int32),
        pltpu.VMEM((NBUF, BCH, NUM_BANDS, D_MODEL), jnp.float32),
        pltpu.SemaphoreType.DMA,
    ],
)
def _lookup(table_hbm, idx_hbm, out_hbm, tab_v, idx_v, buf, ssem):
    wid = lax.axis_index("s") * NC + lax.axis_index("c")
    bbase = wid * BPW
    GROWS = 2 * BCH * NUM_BANDS

    pltpu.sync_copy(table_hbm, tab_v)
    pltpu.sync_copy(idx_hbm.at[pl.ds(bbase * NUM_BANDS, BPW * NUM_BANDS)], idx_v)

    def wait_scatter():
        pltpu.make_async_copy(
            buf.at[0], out_hbm.at[pl.ds(0, BCH)], ssem
        ).wait()

    def group(g, _):
        o = g * GROWS
        iv0 = idx_v[pl.ds(o, LANES)]
        iv1 = idx_v[pl.ds(o + 16, LANES)]
        iv2 = idx_v[pl.ds(o + 24, LANES)]

        for h in range(2):
            t = 2 * g + h
            slot = lax.rem(t, NBUF)

            @pl.when(t >= NBUF)
            def _():
                wait_scatter()

            for r in range(BCH * NUM_BANDS):
                rr = h * BCH * NUM_BANDS + r
                if rr < 16:
                    i = iv0[rr]
                elif rr < 32:
                    i = iv1[rr - 16]
                else:
                    i = iv2[rr - 24]
                base = i * D_MODEL
                bb, j = divmod(r, NUM_BANDS)
                for k in range(NVEC):
                    buf[slot, bb, j, pl.ds(k * LANES, LANES)] = (
                        tab_v[pl.ds(base + k * LANES, LANES)]
                    )

            pltpu.async_copy(
                buf.at[slot], out_hbm.at[pl.ds(bbase + t * BCH, BCH)], ssem
            )
        return 0

    lax.fori_loop(0, NGRP, group, 0)
    for _ in range(NBUF):
        wait_scatter()


def kernel(band_indices, emb_table, freq_w, freq_b):
    table = _combine(emb_table, freq_w.T, freq_b).reshape(NUM_BANDS * D_MODEL)
    idx = band_indices.reshape(BATCH * NUM_BANDS)
    return _lookup(table, idx)


# speedup vs baseline: 2.1622x; 1.2680x over previous
import functools

import jax
import jax.numpy as jnp
from jax import lax
from jax.experimental import pallas as pl
from jax.experimental.pallas import tpu as pltpu
from jax.experimental.pallas import tpu_sc as plsc

D_MODEL = 1024
NUM_BANDS = 5
BATCH = 16384

NC, NS = 2, 16
NW = NC * NS
BPW = BATCH // NW
BCH = 4
NGRP = BPW // (2 * BCH)
NBUF = 3
LANES = 16
NVEC = D_MODEL // LANES

_LO = (0.5, 4.0, 8.0, 13.0, 30.0)
_HI = (4.0, 8.0, 13.0, 30.0, 100.0)


def _combine_body(emb_ref, wt_ref, b_ref, lo_ref, hi_ref, out_ref):
    w0 = wt_ref[0:1, :]
    w1 = wt_ref[1:2, :]
    out_ref[:, :] = (
        emb_ref[:, :] + lo_ref[:, :] * w0 + hi_ref[:, :] * w1
        + b_ref[:].reshape(1, D_MODEL)
    )


def _combine(emb_table, freq_wt, freq_b):
    lo = jnp.array(_LO, dtype=jnp.float32).reshape(NUM_BANDS, 1)
    hi = jnp.array(_HI, dtype=jnp.float32).reshape(NUM_BANDS, 1)
    return pl.pallas_call(
        _combine_body,
        out_shape=jax.ShapeDtypeStruct((NUM_BANDS, D_MODEL), jnp.float32),
    )(emb_table, freq_wt, freq_b, lo, hi)


_MESH = plsc.VectorSubcoreMesh(core_axis_name="c", subcore_axis_name="s")


@functools.partial(
    pl.kernel,
    out_type=jax.ShapeDtypeStruct((BATCH, NUM_BANDS, D_MODEL), jnp.float32),
    mesh=_MESH,
    scratch_types=[
        pltpu.VMEM((NUM_BANDS * D_MODEL,), jnp.float32),
        pltpu.VMEM((BPW * NUM_BANDS,), jnp.int32),
        pltpu.VMEM((NBUF, BCH, NUM_BANDS, D_MODEL), jnp.float32),
        pltpu.SemaphoreType.DMA,
    ],
)
def _lookup(table_hbm, idx_hbm, out_hbm, tab_v, idx_v, buf, ssem):
    wid = lax.axis_index("s") * NC + lax.axis_index("c")
    bbase = wid * BPW
    GROWS = 2 * BCH * NUM_BANDS

    pltpu.sync_copy(table_hbm, tab_v)
    pltpu.sync_copy(idx_hbm.at[pl.ds(bbase * NUM_BANDS, BPW * NUM_BANDS)], idx_v)

    def wait_scatter():
        pltpu.make_async_copy(
            buf.at[0], out_hbm.at[pl.ds(0, BCH)], ssem
        ).wait()

    def group(g, _):
        o = g * GROWS
        iv0 = idx_v[pl.ds(o, LANES)]
        iv1 = idx_v[pl.ds(o + 16, LANES)]
        iv2 = idx_v[pl.ds(o + 24, LANES)]

        for h in range(2):
            t = 2 * g + h
            slot = lax.rem(t, NBUF)

            @pl.when(t >= NBUF)
            def _():
                wait_scatter()

            for r in range(BCH * NUM_BANDS):
                rr = h * BCH * NUM_BANDS + r
                if rr < 16:
                    i = iv0[rr]
                elif rr < 32:
                    i = iv1[rr - 16]
                else:
                    i = iv2[rr - 24]
                base = i * D_MODEL
                bb, j = divmod(r, NUM_BANDS)
                G = 8
                for k0 in range(0, NVEC, G):
                    vs = [
                        tab_v[pl.ds(base + (k0 + u) * LANES, LANES)]
                        for u in range(G)
                    ]
                    for u in range(G):
                        buf[slot, bb, j, pl.ds((k0 + u) * LANES, LANES)] = vs[u]

            pltpu.async_copy(
                buf.at[slot], out_hbm.at[pl.ds(bbase + t * BCH, BCH)], ssem
            )
        return 0

    lax.fori_loop(0, NGRP, group, 0)
    for _ in range(NBUF):
        wait_scatter()


def kernel(band_indices, emb_table, freq_w, freq_b):
    table = _combine(emb_table, freq_w.T, freq_b).reshape(NUM_BANDS * D_MODEL)
    idx = band_indices.reshape(BATCH * NUM_BANDS)
    return _lookup(table, idx)
